# Initial kernel scaffold; baseline (speedup 1.0000x reference)
import jax
import jax.numpy as jnp
from jax.experimental import pallas as pl


def kernel(x, W_skip, b_skip, Wl1, bl1, Wr1, Wl2, bl2, Wr2, Wl3, bl3, Wr3, a, edge_index):
    raise NotImplementedError("write your pallas kernel here")


# speedup vs baseline: 3.3423x; 3.3423x over previous
import functools

import jax
import jax.numpy as jnp
from jax import lax
from jax.experimental import pallas as pl
from jax.experimental.pallas import tpu as pltpu
from jax.experimental.pallas import tpu_sc as plsc

N = 10000
E = 160000
D = 256
H = 128
NC = 2
NS = 16
K = 80
EPW = E // NS
NCHUNK = EPW // K
NP = 10240
RPW = NP // NS

_mesh = plsc.VectorSubcoreMesh(core_axis_name="c", subcore_axis_name="s")


@functools.partial(
    pl.kernel,
    out_type=(
        jax.ShapeDtypeStruct((NP, H), jnp.float32),
        jax.ShapeDtypeStruct((NP, H), jnp.float32),
    ),
    mesh=_mesh,
    scratch_types=[
        pltpu.VMEM_SHARED((NP, H), jnp.float32),
        pltpu.VMEM((K,), jnp.int32),
        pltpu.VMEM((K,), jnp.int32),
        pltpu.VMEM((K, H), jnp.float32),
        pltpu.SemaphoreType.DMA,
    ],
)
def _sc_agg(zA, zB, src, dst, zeros, outA, outB, acc, idx_s, idx_d, rows, sem):
    c = lax.axis_index("c")
    s = lax.axis_index("s")
    r0 = pl.multiple_of(s * RPW, 8)
    pltpu.sync_copy(zeros.at[pl.ds(r0, RPW)], acc.at[pl.ds(r0, RPW)])
    plsc.subcore_barrier()

    def make_chunk(z_ref):
        def chunk(j, carry):
            base = pl.multiple_of(s * EPW + j * K, 8)
            pltpu.sync_copy(src.at[pl.ds(base, K)], idx_s)
            pltpu.sync_copy(dst.at[pl.ds(base, K)], idx_d)
            pltpu.async_copy(z_ref.at[idx_s], rows, sem).wait()
            pltpu.sync_copy(rows, acc.at[idx_d], add=True)
            return carry
        return chunk

    @pl.when(c == 0)
    def _():
        lax.fori_loop(0, NCHUNK, make_chunk(zA), 0)

    @pl.when(c == 1)
    def _():
        lax.fori_loop(0, NCHUNK, make_chunk(zB), 0)

    plsc.subcore_barrier()

    @pl.when(c == 0)
    def _():
        pltpu.sync_copy(acc.at[pl.ds(r0, RPW)], outA.at[pl.ds(r0, RPW)])

    @pl.when(c == 1)
    def _():
        pltpu.sync_copy(acc.at[pl.ds(r0, RPW)], outB.at[pl.ds(r0, RPW)])


KC = 40
EPW2 = E // (NC * NS)
NCHUNK2 = EPW2 // KC


@functools.partial(
    pl.kernel,
    out_type=jax.ShapeDtypeStruct((2 * NP, H), jnp.float32),
    mesh=_mesh,
    scratch_types=[
        pltpu.VMEM_SHARED((NP, H), jnp.float32),
        pltpu.VMEM((KC,), jnp.int32),
        pltpu.VMEM((KC, H), jnp.float32),
        pltpu.SemaphoreType.DMA,
    ],
)
def _sc_counts(dst, zeros, ones128, out, acc, idx_d, ones_v, sem):
    c = lax.axis_index("c")
    s = lax.axis_index("s")
    r0 = pl.multiple_of(s * RPW, 8)
    pltpu.sync_copy(zeros.at[pl.ds(r0, RPW)], acc.at[pl.ds(r0, RPW)])
    pltpu.sync_copy(ones128, ones_v)
    plsc.subcore_barrier()

    def chunk(j, carry):
        base = pl.multiple_of((c * NS + s) * EPW2 + j * KC, 8)
        pltpu.sync_copy(dst.at[pl.ds(base, KC)], idx_d)
        pltpu.sync_copy(ones_v, acc.at[idx_d], add=True)
        return carry

    lax.fori_loop(0, NCHUNK2, chunk, 0)
    plsc.subcore_barrier()
    w0 = pl.multiple_of(c * NP + s * RPW, 8)
    pltpu.sync_copy(acc.at[pl.ds(r0, RPW)], out.at[pl.ds(w0, RPW)])


R = 1024
GRID = NP // R

_row_spec_h = pl.BlockSpec((R, H), lambda i: (i, 0))
_row_spec_d = pl.BlockSpec((R, D), lambda i: (i, 0))
_cnt_spec_a = pl.BlockSpec((R, H), lambda i: (i, 0))
_cnt_spec_b = pl.BlockSpec((R, H), lambda i: (i + NP // R, 0))
_w_hd = pl.BlockSpec((H, D), lambda i: (0, 0))
_w_dd = pl.BlockSpec((D, D), lambda i: (0, 0))
_b_spec = pl.BlockSpec((1, D), lambda i: (0, 0))
_a_spec = pl.BlockSpec(memory_space=pltpu.SMEM)


def _prelu(v, a):
    return jnp.where(v >= 0, v, a * v)


def _mean_term(sA_ref, sB_ref, cnt0_ref, cnt1_ref, WlaT_ref, WlbT_ref, bl_ref):
    cnt = cnt0_ref[:, 0:1] + cnt1_ref[:, 0:1]
    inv = 1.0 / jnp.maximum(cnt, 1.0)
    s = (
        jnp.dot(sA_ref[...], WlaT_ref[...], preferred_element_type=jnp.float32)
        + jnp.dot(sB_ref[...], WlbT_ref[...], preferred_element_type=jnp.float32)
    )
    return s * inv + bl_ref[...]


def _tc1_body(x_ref, sA_ref, sB_ref, cnt0_ref, cnt1_ref, WsT_ref, bs_ref,
              WlaT_ref, WlbT_ref, bl_ref, WrT_ref, a_ref, outA_ref, outB_ref):
    a = a_ref[0]
    x = x_ref[...]
    root = jnp.dot(x, WrT_ref[...], preferred_element_type=jnp.float32)
    h1 = _prelu(_mean_term(sA_ref, sB_ref, cnt0_ref, cnt1_ref, WlaT_ref,
                           WlbT_ref, bl_ref) + root, a)
    z2 = jnp.dot(x, WsT_ref[...], preferred_element_type=jnp.float32) \
        + bs_ref[...] + h1
    outA_ref[...] = z2[:, :H]
    outB_ref[...] = z2[:, H:]


_tc1 = pl.pallas_call(
    _tc1_body,
    grid=(GRID,),
    in_specs=[_row_spec_d, _row_spec_h, _row_spec_h, _cnt_spec_a, _cnt_spec_b,
              _w_dd, _b_spec, _w_hd, _w_hd, _b_spec, _w_dd, _a_spec],
    out_specs=(_row_spec_h, _row_spec_h),
    out_shape=(
        jax.ShapeDtypeStruct((N, H), jnp.float32),
        jax.ShapeDtypeStruct((N, H), jnp.float32),
    ),
)


def _tc23_body(residual, zA_ref, zB_ref, sA_ref, sB_ref, cnt0_ref, cnt1_ref,
               WlaT_ref, WlbT_ref, bl_ref, WraT_ref, WrbT_ref, a_ref,
               outA_ref, outB_ref):
    a = a_ref[0]
    root = (
        jnp.dot(zA_ref[...], WraT_ref[...], preferred_element_type=jnp.float32)
        + jnp.dot(zB_ref[...], WrbT_ref[...], preferred_element_type=jnp.float32)
    )
    h = _prelu(_mean_term(sA_ref, sB_ref, cnt0_ref, cnt1_ref, WlaT_ref,
                          WlbT_ref, bl_ref) + root, a)
    if residual:
        outA_ref[...] = zA_ref[...] + h[:, :H]
        outB_ref[...] = zB_ref[...] + h[:, H:]
    else:
        outA_ref[...] = h[:, :H]
        outB_ref[...] = h[:, H:]


def _make_tc23(residual):
    return pl.pallas_call(
        functools.partial(_tc23_body, residual),
        grid=(GRID,),
        in_specs=[_row_spec_h, _row_spec_h, _row_spec_h, _row_spec_h,
                  _cnt_spec_a, _cnt_spec_b, _w_hd, _w_hd, _b_spec, _w_hd,
                  _w_hd, _a_spec],
        out_specs=(_row_spec_h, _row_spec_h),
        out_shape=(
            jax.ShapeDtypeStruct((N, H), jnp.float32),
            jax.ShapeDtypeStruct((N, H), jnp.float32),
        ),
    )


_tc2 = _make_tc23(True)
_tc3 = _make_tc23(False)


def kernel(x, W_skip, b_skip, Wl1, bl1, Wr1, Wl2, bl2, Wr2, Wl3, bl3, Wr3, a,
           edge_index):
    f32 = jnp.float32
    src = edge_index[0].astype(jnp.int32)
    dst = edge_index[1].astype(jnp.int32)

    xA = x[:, :H]
    xB = x[:, H:]
    zeros = jnp.zeros((NP, H), f32)
    ones128 = jnp.ones((KC, H), f32)

    WsT = W_skip.T
    Wr1T = Wr1.T
    bs2 = b_skip.reshape(1, D)
    bl1_2 = bl1.reshape(1, D)
    bl2_2 = bl2.reshape(1, D)
    bl3_2 = bl3.reshape(1, D)
    Wl1aT, Wl1bT = Wl1[:, :H].T, Wl1[:, H:].T
    Wl2aT, Wl2bT = Wl2[:, :H].T, Wl2[:, H:].T
    Wl3aT, Wl3bT = Wl3[:, :H].T, Wl3[:, H:].T
    Wr2aT, Wr2bT = Wr2[:, :H].T, Wr2[:, H:].T
    Wr3aT, Wr3bT = Wr3[:, :H].T, Wr3[:, H:].T
    a1 = a.reshape(1).astype(f32)

    cntS = _sc_counts(dst, zeros, ones128)

    sA, sB = _sc_agg(xA, xB, src, dst, zeros)
    z2A, z2B = _tc1(x, sA, sB, cntS, cntS, WsT, bs2, Wl1aT, Wl1bT, bl1_2,
                    Wr1T, a1)

    sA, sB = _sc_agg(z2A, z2B, src, dst, zeros)
    z3A, z3B = _tc2(z2A, z2B, sA, sB, cntS, cntS, Wl2aT, Wl2bT, bl2_2, Wr2aT,
                    Wr2bT, a1)

    sA, sB = _sc_agg(z3A, z3B, src, dst, zeros)
    h3A, h3B = _tc3(z3A, z3B, sA, sB, cntS, cntS, Wl3aT, Wl3bT, bl3_2, Wr3aT,
                    Wr3bT, a1)

    return jnp.concatenate([h3A, h3B], axis=1)


# speedup vs baseline: 5.4880x; 1.6420x over previous
import functools

import jax
import jax.numpy as jnp
from jax import lax
from jax.experimental import pallas as pl
from jax.experimental.pallas import tpu as pltpu
from jax.experimental.pallas import tpu_sc as plsc

N = 10000
E = 160000
D = 256
H = 128
NC = 2
NS = 16
K = 40
EPW = E // NS
NCHUNK = EPW // K
NP = 10240
RPW = NP // NS

_mesh = plsc.VectorSubcoreMesh(core_axis_name="c", subcore_axis_name="s")


G = 5


@functools.partial(
    pl.kernel,
    out_type=(
        jax.ShapeDtypeStruct((NP, H), jnp.float32),
        jax.ShapeDtypeStruct((NP, H), jnp.float32),
    ),
    mesh=_mesh,
    scratch_types=[
        pltpu.VMEM_SHARED((NP, H), jnp.float32),
        pltpu.VMEM((G, K), jnp.int32),
        pltpu.VMEM((G, K), jnp.int32),
        pltpu.VMEM((G, K, H), jnp.float32),
        [pltpu.SemaphoreType.DMA] * G,
        [pltpu.SemaphoreType.DMA] * G,
        [pltpu.SemaphoreType.DMA] * G,
        pltpu.SemaphoreType.DMA,
    ],
)
def _sc_agg(zA, zB, src, dst, zeros, outA, outB, acc, idx_s, idx_d, rows,
            isems, dsems, gsems, ssem):
    c = lax.axis_index("c")
    s = lax.axis_index("s")
    r0 = pl.multiple_of(s * RPW, 8)
    pltpu.sync_copy(zeros.at[pl.ds(r0, RPW)], acc.at[pl.ds(r0, RPW)])
    plsc.subcore_barrier()

    def make_outer(z_ref):
        def outer(jo, carry):
            j0 = jo * G
            ids_ = []
            for b in range(G):
                base = pl.multiple_of(s * EPW + (j0 + b) * K, 8)
                ids_.append((
                    pltpu.async_copy(src.at[pl.ds(base, K)], idx_s.at[b],
                                     isems[b]),
                    pltpu.async_copy(dst.at[pl.ds(base, K)], idx_d.at[b],
                                     dsems[b]),
                ))
            gds = []
            for b in range(G):
                ids_[b][0].wait()
                gds.append(pltpu.async_copy(z_ref.at[idx_s.at[b]], rows.at[b],
                                            gsems[b]))
            sds = []
            for b in range(G):
                gds[b].wait()
                ids_[b][1].wait()
                sds.append(pltpu.async_copy(rows.at[b], acc.at[idx_d.at[b]],
                                            ssem, add=True))
            for d in sds:
                d.wait()
            return carry
        return outer

    @pl.when(c == 0)
    def _():
        lax.fori_loop(0, NCHUNK // G, make_outer(zA), 0)

    @pl.when(c == 1)
    def _():
        lax.fori_loop(0, NCHUNK // G, make_outer(zB), 0)

    plsc.subcore_barrier()

    @pl.when(c == 0)
    def _():
        pltpu.sync_copy(acc.at[pl.ds(r0, RPW)], outA.at[pl.ds(r0, RPW)])

    @pl.when(c == 1)
    def _():
        pltpu.sync_copy(acc.at[pl.ds(r0, RPW)], outB.at[pl.ds(r0, RPW)])


KC = 40
EPW2 = E // (NC * NS)
NCHUNK2 = EPW2 // KC


@functools.partial(
    pl.kernel,
    out_type=jax.ShapeDtypeStruct((2 * NP, H), jnp.float32),
    mesh=_mesh,
    scratch_types=[
        pltpu.VMEM_SHARED((NP, H), jnp.float32),
        pltpu.VMEM((KC,), jnp.int32),
        pltpu.VMEM((KC, H), jnp.float32),
        pltpu.SemaphoreType.DMA,
    ],
)
def _sc_counts(dst, zeros, ones128, out, acc, idx_d, ones_v, sem):
    c = lax.axis_index("c")
    s = lax.axis_index("s")
    r0 = pl.multiple_of(s * RPW, 8)
    pltpu.sync_copy(zeros.at[pl.ds(r0, RPW)], acc.at[pl.ds(r0, RPW)])
    pltpu.sync_copy(ones128, ones_v)
    plsc.subcore_barrier()

    def chunk(j, carry):
        base = pl.multiple_of((c * NS + s) * EPW2 + j * KC, 8)
        pltpu.sync_copy(dst.at[pl.ds(base, KC)], idx_d)
        pltpu.sync_copy(ones_v, acc.at[idx_d], add=True)
        return carry

    lax.fori_loop(0, NCHUNK2, chunk, 0)
    plsc.subcore_barrier()
    w0 = pl.multiple_of(c * NP + s * RPW, 8)
    pltpu.sync_copy(acc.at[pl.ds(r0, RPW)], out.at[pl.ds(w0, RPW)])


R = 1024
GRID = NP // R

_row_spec_h = pl.BlockSpec((R, H), lambda i: (i, 0))
_row_spec_d = pl.BlockSpec((R, D), lambda i: (i, 0))
_cnt_spec_a = pl.BlockSpec((R, H), lambda i: (i, 0))
_cnt_spec_b = pl.BlockSpec((R, H), lambda i: (i + NP // R, 0))
_w_hd = pl.BlockSpec((H, D), lambda i: (0, 0))
_w_dd = pl.BlockSpec((D, D), lambda i: (0, 0))
_b_spec = pl.BlockSpec((1, D), lambda i: (0, 0))
_a_spec = pl.BlockSpec(memory_space=pltpu.SMEM)


def _prelu(v, a):
    return jnp.where(v >= 0, v, a * v)


def _mean_term(sA_ref, sB_ref, cnt0_ref, cnt1_ref, WlaT_ref, WlbT_ref, bl_ref):
    cnt = cnt0_ref[:, 0:1] + cnt1_ref[:, 0:1]
    inv = 1.0 / jnp.maximum(cnt, 1.0)
    s = (
        jnp.dot(sA_ref[...], WlaT_ref[...], preferred_element_type=jnp.float32)
        + jnp.dot(sB_ref[...], WlbT_ref[...], preferred_element_type=jnp.float32)
    )
    return s * inv + bl_ref[...]


def _tc1_body(x_ref, sA_ref, sB_ref, cnt0_ref, cnt1_ref, WsT_ref, bs_ref,
              WlaT_ref, WlbT_ref, bl_ref, WrT_ref, a_ref, outA_ref, outB_ref):
    a = a_ref[0]
    x = x_ref[...]
    root = jnp.dot(x, WrT_ref[...], preferred_element_type=jnp.float32)
    h1 = _prelu(_mean_term(sA_ref, sB_ref, cnt0_ref, cnt1_ref, WlaT_ref,
                           WlbT_ref, bl_ref) + root, a)
    z2 = jnp.dot(x, WsT_ref[...], preferred_element_type=jnp.float32) \
        + bs_ref[...] + h1
    outA_ref[...] = z2[:, :H]
    outB_ref[...] = z2[:, H:]


_tc1 = pl.pallas_call(
    _tc1_body,
    grid=(GRID,),
    in_specs=[_row_spec_d, _row_spec_h, _row_spec_h, _cnt_spec_a, _cnt_spec_b,
              _w_dd, _b_spec, _w_hd, _w_hd, _b_spec, _w_dd, _a_spec],
    out_specs=(_row_spec_h, _row_spec_h),
    out_shape=(
        jax.ShapeDtypeStruct((N, H), jnp.float32),
        jax.ShapeDtypeStruct((N, H), jnp.float32),
    ),
)


def _tc23_body(residual, zA_ref, zB_ref, sA_ref, sB_ref, cnt0_ref, cnt1_ref,
               WlaT_ref, WlbT_ref, bl_ref, WraT_ref, WrbT_ref, a_ref,
               outA_ref, outB_ref):
    a = a_ref[0]
    root = (
        jnp.dot(zA_ref[...], WraT_ref[...], preferred_element_type=jnp.float32)
        + jnp.dot(zB_ref[...], WrbT_ref[...], preferred_element_type=jnp.float32)
    )
    h = _prelu(_mean_term(sA_ref, sB_ref, cnt0_ref, cnt1_ref, WlaT_ref,
                          WlbT_ref, bl_ref) + root, a)
    if residual:
        outA_ref[...] = zA_ref[...] + h[:, :H]
        outB_ref[...] = zB_ref[...] + h[:, H:]
    else:
        outA_ref[...] = h[:, :H]
        outB_ref[...] = h[:, H:]


def _make_tc23(residual):
    return pl.pallas_call(
        functools.partial(_tc23_body, residual),
        grid=(GRID,),
        in_specs=[_row_spec_h, _row_spec_h, _row_spec_h, _row_spec_h,
                  _cnt_spec_a, _cnt_spec_b, _w_hd, _w_hd, _b_spec, _w_hd,
                  _w_hd, _a_spec],
        out_specs=(_row_spec_h, _row_spec_h),
        out_shape=(
            jax.ShapeDtypeStruct((N, H), jnp.float32),
            jax.ShapeDtypeStruct((N, H), jnp.float32),
        ),
    )


_tc2 = _make_tc23(True)
_tc3 = _make_tc23(False)


def kernel(x, W_skip, b_skip, Wl1, bl1, Wr1, Wl2, bl2, Wr2, Wl3, bl3, Wr3, a,
           edge_index):
    f32 = jnp.float32
    src = edge_index[0].astype(jnp.int32)
    dst = edge_index[1].astype(jnp.int32)

    xA = x[:, :H]
    xB = x[:, H:]
    zeros = jnp.zeros((NP, H), f32)
    ones128 = jnp.ones((KC, H), f32)

    WsT = W_skip.T
    Wr1T = Wr1.T
    bs2 = b_skip.reshape(1, D)
    bl1_2 = bl1.reshape(1, D)
    bl2_2 = bl2.reshape(1, D)
    bl3_2 = bl3.reshape(1, D)
    Wl1aT, Wl1bT = Wl1[:, :H].T, Wl1[:, H:].T
    Wl2aT, Wl2bT = Wl2[:, :H].T, Wl2[:, H:].T
    Wl3aT, Wl3bT = Wl3[:, :H].T, Wl3[:, H:].T
    Wr2aT, Wr2bT = Wr2[:, :H].T, Wr2[:, H:].T
    Wr3aT, Wr3bT = Wr3[:, :H].T, Wr3[:, H:].T
    a1 = a.reshape(1).astype(f32)

    cntS = _sc_counts(dst, zeros, ones128)

    sA, sB = _sc_agg(xA, xB, src, dst, zeros)
    z2A, z2B = _tc1(x, sA, sB, cntS, cntS, WsT, bs2, Wl1aT, Wl1bT, bl1_2,
                    Wr1T, a1)

    sA, sB = _sc_agg(z2A, z2B, src, dst, zeros)
    z3A, z3B = _tc2(z2A, z2B, sA, sB, cntS, cntS, Wl2aT, Wl2bT, bl2_2, Wr2aT,
                    Wr2bT, a1)

    sA, sB = _sc_agg(z3A, z3B, src, dst, zeros)
    h3A, h3B = _tc3(z3A, z3B, sA, sB, cntS, cntS, Wl3aT, Wl3bT, bl3_2, Wr3aT,
                    Wr3bT, a1)

    return jnp.concatenate([h3A, h3B], axis=1)


# speedup vs baseline: 6.4422x; 1.1739x over previous
import functools

import jax
import jax.numpy as jnp
from jax import lax
from jax.experimental import pallas as pl
from jax.experimental.pallas import tpu as pltpu
from jax.experimental.pallas import tpu_sc as plsc

N = 10000
E = 160000
D = 256
H = 128
NC = 2
NS = 16
K = 40
EPW = E // NS
NCHUNK = EPW // K
NP = 10240
RPW = NP // NS

_mesh = plsc.VectorSubcoreMesh(core_axis_name="c", subcore_axis_name="s")


G = 5


@functools.partial(
    pl.kernel,
    out_type=(
        jax.ShapeDtypeStruct((NP, H), jnp.float32),
        jax.ShapeDtypeStruct((NP, H), jnp.float32),
    ),
    mesh=_mesh,
    scratch_types=[
        pltpu.VMEM_SHARED((NP, H), jnp.float32),
        pltpu.VMEM((G, K), jnp.int32),
        pltpu.VMEM((G, K), jnp.int32),
        pltpu.VMEM((G, K, H), jnp.float32),
        [pltpu.SemaphoreType.DMA] * G,
        [pltpu.SemaphoreType.DMA] * G,
        [pltpu.SemaphoreType.DMA] * G,
        pltpu.SemaphoreType.DMA,
    ],
)
def _sc_agg(zA, zB, src, dst, zeros, outA, outB, acc, idx_s, idx_d, rows,
            isems, dsems, gsems, ssem):
    c = lax.axis_index("c")
    s = lax.axis_index("s")
    r0 = pl.multiple_of(s * RPW, 8)
    pltpu.sync_copy(zeros.at[pl.ds(r0, RPW)], acc.at[pl.ds(r0, RPW)])
    plsc.subcore_barrier()

    def make_outer(z_ref):
        def outer(jo, carry):
            j0 = jo * G
            ids_ = []
            for b in range(G):
                base = pl.multiple_of(s * EPW + (j0 + b) * K, 8)
                ids_.append((
                    pltpu.async_copy(src.at[pl.ds(base, K)], idx_s.at[b],
                                     isems[b]),
                    pltpu.async_copy(dst.at[pl.ds(base, K)], idx_d.at[b],
                                     dsems[b]),
                ))
            gds = []
            for b in range(G):
                ids_[b][0].wait()
                gds.append(pltpu.async_copy(z_ref.at[idx_s.at[b]], rows.at[b],
                                            gsems[b]))
            sds = []
            for b in range(G):
                gds[b].wait()
                ids_[b][1].wait()
                sds.append(pltpu.async_copy(rows.at[b], acc.at[idx_d.at[b]],
                                            ssem, add=True))
            for d in sds:
                d.wait()
            return carry
        return outer

    @pl.when(c == 0)
    def _():
        lax.fori_loop(0, NCHUNK // G, make_outer(zA), 0)

    @pl.when(c == 1)
    def _():
        lax.fori_loop(0, NCHUNK // G, make_outer(zB), 0)

    plsc.subcore_barrier()

    @pl.when(c == 0)
    def _():
        pltpu.sync_copy(acc.at[pl.ds(r0, RPW)], outA.at[pl.ds(r0, RPW)])

    @pl.when(c == 1)
    def _():
        pltpu.sync_copy(acc.at[pl.ds(r0, RPW)], outB.at[pl.ds(r0, RPW)])


EPW2 = E // (NC * NS)
NVEC2 = EPW2 // 16


@functools.partial(
    pl.kernel,
    out_type=jax.ShapeDtypeStruct((NC * NS, NP), jnp.float32),
    mesh=_mesh,
    scratch_types=[
        pltpu.VMEM((NP,), jnp.float32),
        pltpu.VMEM((EPW2,), jnp.int32),
        pltpu.SemaphoreType.DMA,
    ],
    compiler_params=pltpu.CompilerParams(needs_layout_passes=False),
)
def _sc_counts(dst, out, hist, idxall, sem):
    c = lax.axis_index("c")
    s = lax.axis_index("s")
    w = c * NS + s
    base = pl.multiple_of(w * EPW2, 8)
    pltpu.sync_copy(dst.at[pl.ds(base, EPW2)], idxall)

    zero16 = jnp.zeros((16,), jnp.float32)

    def zero_step(i, carry):
        hist[pl.ds(i * 16, 16)] = zero16
        return carry

    lax.fori_loop(0, NP // 16, zero_step, 0)

    ones = jnp.ones((16,), jnp.float32)

    def add_step(i, carry):
        idx = idxall[pl.ds(i * 16, 16)]
        plsc.addupdate_scatter(hist, [idx], ones)
        return carry

    lax.fori_loop(0, NVEC2, add_step, 0)
    tail = idxall[pl.ds(NVEC2 * 16 - 8, 16)]
    mask = lax.iota(jnp.int32, 16) >= 8
    plsc.addupdate_scatter(hist, [tail], ones, mask=mask)

    pltpu.sync_copy(hist, out.at[w])


def _inv_body(cnt_ref, out_ref):
    t = jnp.transpose(cnt_ref[...])
    cnt = jnp.sum(t, axis=1, keepdims=True)
    inv = 1.0 / jnp.maximum(cnt, 1.0)
    out_ref[...] = jnp.broadcast_to(inv, out_ref.shape)


R = 1024
GRID = NP // R

_row_spec_h = pl.BlockSpec((R, H), lambda i: (i, 0))
_row_spec_d = pl.BlockSpec((R, D), lambda i: (i, 0))
_cnt_spec = pl.BlockSpec((R, 8), lambda i: (i, 0))
_w_hd = pl.BlockSpec((H, D), lambda i: (0, 0))
_w_dd = pl.BlockSpec((D, D), lambda i: (0, 0))
_b_spec = pl.BlockSpec((1, D), lambda i: (0, 0))
_a_spec = pl.BlockSpec(memory_space=pltpu.SMEM)


_tc_inv = pl.pallas_call(
    _inv_body,
    grid=(GRID,),
    in_specs=[pl.BlockSpec((NC * NS, R), lambda i: (0, i))],
    out_specs=pl.BlockSpec((R, 8), lambda i: (i, 0)),
    out_shape=jax.ShapeDtypeStruct((NP, 8), jnp.float32),
)


def _prelu(v, a):
    return jnp.where(v >= 0, v, a * v)


def _mean_term(sA_ref, sB_ref, inv_ref, WlaT_ref, WlbT_ref, bl_ref):
    inv = inv_ref[:, 0:1]
    s = (
        jnp.dot(sA_ref[...], WlaT_ref[...], preferred_element_type=jnp.float32)
        + jnp.dot(sB_ref[...], WlbT_ref[...], preferred_element_type=jnp.float32)
    )
    return s * inv + bl_ref[...]


def _tc1_body(x_ref, sA_ref, sB_ref, inv_ref, WsT_ref, bs_ref,
              WlaT_ref, WlbT_ref, bl_ref, WrT_ref, a_ref, outA_ref, outB_ref):
    a = a_ref[0]
    x = x_ref[...]
    root = jnp.dot(x, WrT_ref[...], preferred_element_type=jnp.float32)
    h1 = _prelu(_mean_term(sA_ref, sB_ref, inv_ref, WlaT_ref,
                           WlbT_ref, bl_ref) + root, a)
    z2 = jnp.dot(x, WsT_ref[...], preferred_element_type=jnp.float32) \
        + bs_ref[...] + h1
    outA_ref[...] = z2[:, :H]
    outB_ref[...] = z2[:, H:]


_tc1 = pl.pallas_call(
    _tc1_body,
    grid=(GRID,),
    in_specs=[_row_spec_d, _row_spec_h, _row_spec_h, _cnt_spec,
              _w_dd, _b_spec, _w_hd, _w_hd, _b_spec, _w_dd, _a_spec],
    out_specs=(_row_spec_h, _row_spec_h),
    out_shape=(
        jax.ShapeDtypeStruct((N, H), jnp.float32),
        jax.ShapeDtypeStruct((N, H), jnp.float32),
    ),
)


def _tc23_body(residual, zA_ref, zB_ref, sA_ref, sB_ref, inv_ref,
               WlaT_ref, WlbT_ref, bl_ref, WraT_ref, WrbT_ref, a_ref,
               outA_ref, outB_ref):
    a = a_ref[0]
    root = (
        jnp.dot(zA_ref[...], WraT_ref[...], preferred_element_type=jnp.float32)
        + jnp.dot(zB_ref[...], WrbT_ref[...], preferred_element_type=jnp.float32)
    )
    h = _prelu(_mean_term(sA_ref, sB_ref, inv_ref, WlaT_ref,
                          WlbT_ref, bl_ref) + root, a)
    if residual:
        outA_ref[...] = zA_ref[...] + h[:, :H]
        outB_ref[...] = zB_ref[...] + h[:, H:]
    else:
        outA_ref[...] = h[:, :H]
        outB_ref[...] = h[:, H:]


def _make_tc23(residual):
    return pl.pallas_call(
        functools.partial(_tc23_body, residual),
        grid=(GRID,),
        in_specs=[_row_spec_h, _row_spec_h, _row_spec_h, _row_spec_h,
                  _cnt_spec, _w_hd, _w_hd, _b_spec, _w_hd,
                  _w_hd, _a_spec],
        out_specs=(_row_spec_h, _row_spec_h),
        out_shape=(
            jax.ShapeDtypeStruct((N, H), jnp.float32),
            jax.ShapeDtypeStruct((N, H), jnp.float32),
        ),
    )


_tc2 = _make_tc23(True)
_tc3 = _make_tc23(False)


def kernel(x, W_skip, b_skip, Wl1, bl1, Wr1, Wl2, bl2, Wr2, Wl3, bl3, Wr3, a,
           edge_index):
    f32 = jnp.float32
    src = edge_index[0].astype(jnp.int32)
    dst = edge_index[1].astype(jnp.int32)

    xA = x[:, :H]
    xB = x[:, H:]
    zeros = jnp.zeros((NP, H), f32)

    WsT = W_skip.T
    Wr1T = Wr1.T
    bs2 = b_skip.reshape(1, D)
    bl1_2 = bl1.reshape(1, D)
    bl2_2 = bl2.reshape(1, D)
    bl3_2 = bl3.reshape(1, D)
    Wl1aT, Wl1bT = Wl1[:, :H].T, Wl1[:, H:].T
    Wl2aT, Wl2bT = Wl2[:, :H].T, Wl2[:, H:].T
    Wl3aT, Wl3bT = Wl3[:, :H].T, Wl3[:, H:].T
    Wr2aT, Wr2bT = Wr2[:, :H].T, Wr2[:, H:].T
    Wr3aT, Wr3bT = Wr3[:, :H].T, Wr3[:, H:].T
    a1 = a.reshape(1).astype(f32)

    cnt32 = _sc_counts(dst)
    inv8 = _tc_inv(cnt32)

    sA, sB = _sc_agg(xA, xB, src, dst, zeros)
    z2A, z2B = _tc1(x, sA, sB, inv8, WsT, bs2, Wl1aT, Wl1bT, bl1_2,
                    Wr1T, a1)

    sA, sB = _sc_agg(z2A, z2B, src, dst, zeros)
    z3A, z3B = _tc2(z2A, z2B, sA, sB, inv8, Wl2aT, Wl2bT, bl2_2, Wr2aT,
                    Wr2bT, a1)

    sA, sB = _sc_agg(z3A, z3B, src, dst, zeros)
    h3A, h3B = _tc3(z3A, z3B, sA, sB, inv8, Wl3aT, Wl3bT, bl3_2, Wr3aT,
                    Wr3bT, a1)

    return jnp.concatenate([h3A, h3B], axis=1)


# speedup vs baseline: 8.1324x; 1.2624x over previous
import functools

import jax
import jax.numpy as jnp
from jax import lax
from jax.experimental import pallas as pl
from jax.experimental.pallas import tpu as pltpu
from jax.experimental.pallas import tpu_sc as plsc

N = 10000
E = 160000
D = 256
H = 128
NC = 2
NS = 16
K = 40
EPW = E // NS
NCHUNK = EPW // K
NP = 10240
RPW = NP // NS

_mesh = plsc.VectorSubcoreMesh(core_axis_name="c", subcore_axis_name="s")


G = 5
NOUTER = NCHUNK // G


@functools.partial(
    pl.kernel,
    out_type=(
        jax.ShapeDtypeStruct((NP, H), jnp.float32),
        jax.ShapeDtypeStruct((NP, H), jnp.float32),
    ),
    mesh=_mesh,
    scratch_types=[
        pltpu.VMEM_SHARED((NP, H), jnp.float32),
        pltpu.VMEM((2, G, K), jnp.int32),
        pltpu.VMEM((2, G, K), jnp.int32),
        pltpu.VMEM((G, K, H), jnp.float32),
        [pltpu.SemaphoreType.DMA] * (2 * G),
        [pltpu.SemaphoreType.DMA] * (2 * G),
        [pltpu.SemaphoreType.DMA] * G,
        [pltpu.SemaphoreType.DMA] * G,
    ],
)
def _sc_agg(zA, zB, src, dst, zeros, outA, outB, acc, idx_s, idx_d, rows,
            isems, dsems, gsems, ssems):
    c = lax.axis_index("c")
    s = lax.axis_index("s")
    r0 = pl.multiple_of(s * RPW, 8)
    pltpu.sync_copy(zeros.at[pl.ds(r0, RPW)], acc.at[pl.ds(r0, RPW)])
    plsc.subcore_barrier()

    def issue_idx(jo, p, b):
        base = pl.multiple_of(s * EPW + (jo * G + b) * K, 8)
        pltpu.async_copy(src.at[pl.ds(base, K)], idx_s.at[p, b],
                         isems[p * G + b])
        pltpu.async_copy(dst.at[pl.ds(base, K)], idx_d.at[p, b],
                         dsems[p * G + b])

    def wait_idx_s(p, b):
        pltpu.make_async_copy(src.at[pl.ds(0, K)], idx_s.at[p, b],
                              isems[p * G + b]).wait()

    def wait_idx_d(p, b):
        pltpu.make_async_copy(dst.at[pl.ds(0, K)], idx_d.at[p, b],
                              dsems[p * G + b]).wait()

    def wait_scatter(p, b):
        pltpu.make_async_copy(rows.at[b], acc.at[idx_d.at[p, b]],
                              ssems[b]).wait()

    def make_body(z_ref):
        def body(jo, p):
            q = 1 - p
            gds = []
            for b in range(G):
                @pl.when(jo > 0)
                def _():
                    wait_scatter(q, b)

                @pl.when(jo + 1 < NOUTER)
                def _():
                    issue_idx(jo + 1, q, b)

                wait_idx_s(p, b)
                gds.append(pltpu.async_copy(z_ref.at[idx_s.at[p, b]],
                                            rows.at[b], gsems[b]))
            for b in range(G):
                gds[b].wait()
                wait_idx_d(p, b)
                pltpu.async_copy(rows.at[b], acc.at[idx_d.at[p, b]],
                                 ssems[b], add=True)
        return body

    def make_outer(z_ref):
        body = make_body(z_ref)

        def outer(jo2, carry):
            body(2 * jo2, 0)
            body(2 * jo2 + 1, 1)
            return carry
        return outer

    def run(z_ref):
        for b in range(G):
            issue_idx(0, 0, b)
        lax.fori_loop(0, NOUTER // 2, make_outer(z_ref), 0)
        for b in range(G):
            wait_scatter(1, b)

    @pl.when(c == 0)
    def _():
        run(zA)

    @pl.when(c == 1)
    def _():
        run(zB)

    plsc.subcore_barrier()

    @pl.when(c == 0)
    def _():
        pltpu.sync_copy(acc.at[pl.ds(r0, RPW)], outA.at[pl.ds(r0, RPW)])

    @pl.when(c == 1)
    def _():
        pltpu.sync_copy(acc.at[pl.ds(r0, RPW)], outB.at[pl.ds(r0, RPW)])


EPW2 = E // (NC * NS)
NVEC2 = EPW2 // 16


@functools.partial(
    pl.kernel,
    out_type=jax.ShapeDtypeStruct((NC * NS, NP), jnp.float32),
    mesh=_mesh,
    scratch_types=[
        pltpu.VMEM((NP,), jnp.float32),
        pltpu.VMEM((EPW2,), jnp.int32),
        pltpu.SemaphoreType.DMA,
    ],
    compiler_params=pltpu.CompilerParams(needs_layout_passes=False),
)
def _sc_counts(dst, out, hist, idxall, sem):
    c = lax.axis_index("c")
    s = lax.axis_index("s")
    w = c * NS + s
    base = pl.multiple_of(w * EPW2, 8)
    pltpu.sync_copy(dst.at[pl.ds(base, EPW2)], idxall)

    zero16 = jnp.zeros((16,), jnp.float32)

    def zero_step(i, carry):
        hist[pl.ds(i * 16, 16)] = zero16
        return carry

    lax.fori_loop(0, NP // 16, zero_step, 0)

    ones = jnp.ones((16,), jnp.float32)

    def add_step(i, carry):
        idx = idxall[pl.ds(i * 16, 16)]
        plsc.addupdate_scatter(hist, [idx], ones)
        return carry

    lax.fori_loop(0, NVEC2, add_step, 0)
    tail = idxall[pl.ds(NVEC2 * 16 - 8, 16)]
    mask = lax.iota(jnp.int32, 16) >= 8
    plsc.addupdate_scatter(hist, [tail], ones, mask=mask)

    pltpu.sync_copy(hist, out.at[w])


def _inv_body(cnt_ref, out_ref):
    t = jnp.transpose(cnt_ref[...])
    cnt = jnp.sum(t, axis=1, keepdims=True)
    inv = 1.0 / jnp.maximum(cnt, 1.0)
    out_ref[...] = jnp.broadcast_to(inv, out_ref.shape)


R = 1024
GRID = NP // R

_row_spec_h = pl.BlockSpec((R, H), lambda i: (i, 0))
_row_spec_d = pl.BlockSpec((R, D), lambda i: (i, 0))
_cnt_spec = pl.BlockSpec((R, 8), lambda i: (i, 0))
_w_hd = pl.BlockSpec((H, D), lambda i: (0, 0))
_w_dd = pl.BlockSpec((D, D), lambda i: (0, 0))
_b_spec = pl.BlockSpec((1, D), lambda i: (0, 0))
_a_spec = pl.BlockSpec(memory_space=pltpu.SMEM)


_tc_inv = pl.pallas_call(
    _inv_body,
    grid=(GRID,),
    in_specs=[pl.BlockSpec((NC * NS, R), lambda i: (0, i))],
    out_specs=pl.BlockSpec((R, 8), lambda i: (i, 0)),
    out_shape=jax.ShapeDtypeStruct((NP, 8), jnp.float32),
)


def _prelu(v, a):
    return jnp.where(v >= 0, v, a * v)


def _mean_term(sA_ref, sB_ref, inv_ref, WlaT_ref, WlbT_ref, bl_ref):
    inv = inv_ref[:, 0:1]
    s = (
        jnp.dot(sA_ref[...], WlaT_ref[...], preferred_element_type=jnp.float32)
        + jnp.dot(sB_ref[...], WlbT_ref[...], preferred_element_type=jnp.float32)
    )
    return s * inv + bl_ref[...]


def _tc1_body(x_ref, sA_ref, sB_ref, inv_ref, WsT_ref, bs_ref,
              WlaT_ref, WlbT_ref, bl_ref, WrT_ref, a_ref, outA_ref, outB_ref):
    a = a_ref[0]
    x = x_ref[...]
    root = jnp.dot(x, WrT_ref[...], preferred_element_type=jnp.float32)
    h1 = _prelu(_mean_term(sA_ref, sB_ref, inv_ref, WlaT_ref,
                           WlbT_ref, bl_ref) + root, a)
    z2 = jnp.dot(x, WsT_ref[...], preferred_element_type=jnp.float32) \
        + bs_ref[...] + h1
    outA_ref[...] = z2[:, :H]
    outB_ref[...] = z2[:, H:]


_tc1 = pl.pallas_call(
    _tc1_body,
    grid=(GRID,),
    in_specs=[_row_spec_d, _row_spec_h, _row_spec_h, _cnt_spec,
              _w_dd, _b_spec, _w_hd, _w_hd, _b_spec, _w_dd, _a_spec],
    out_specs=(_row_spec_h, _row_spec_h),
    out_shape=(
        jax.ShapeDtypeStruct((N, H), jnp.float32),
        jax.ShapeDtypeStruct((N, H), jnp.float32),
    ),
)


def _tc23_body(residual, zA_ref, zB_ref, sA_ref, sB_ref, inv_ref,
               WlaT_ref, WlbT_ref, bl_ref, WraT_ref, WrbT_ref, a_ref,
               outA_ref, outB_ref):
    a = a_ref[0]
    root = (
        jnp.dot(zA_ref[...], WraT_ref[...], preferred_element_type=jnp.float32)
        + jnp.dot(zB_ref[...], WrbT_ref[...], preferred_element_type=jnp.float32)
    )
    h = _prelu(_mean_term(sA_ref, sB_ref, inv_ref, WlaT_ref,
                          WlbT_ref, bl_ref) + root, a)
    if residual:
        outA_ref[...] = zA_ref[...] + h[:, :H]
        outB_ref[...] = zB_ref[...] + h[:, H:]
    else:
        outA_ref[...] = h[:, :H]
        outB_ref[...] = h[:, H:]


def _make_tc23(residual):
    return pl.pallas_call(
        functools.partial(_tc23_body, residual),
        grid=(GRID,),
        in_specs=[_row_spec_h, _row_spec_h, _row_spec_h, _row_spec_h,
                  _cnt_spec, _w_hd, _w_hd, _b_spec, _w_hd,
                  _w_hd, _a_spec],
        out_specs=(_row_spec_h, _row_spec_h),
        out_shape=(
            jax.ShapeDtypeStruct((N, H), jnp.float32),
            jax.ShapeDtypeStruct((N, H), jnp.float32),
        ),
    )


_tc2 = _make_tc23(True)
_tc3 = _make_tc23(False)


def kernel(x, W_skip, b_skip, Wl1, bl1, Wr1, Wl2, bl2, Wr2, Wl3, bl3, Wr3, a,
           edge_index):
    f32 = jnp.float32
    src = edge_index[0].astype(jnp.int32)
    dst = edge_index[1].astype(jnp.int32)

    xA = x[:, :H]
    xB = x[:, H:]
    zeros = jnp.zeros((NP, H), f32)

    WsT = W_skip.T
    Wr1T = Wr1.T
    bs2 = b_skip.reshape(1, D)
    bl1_2 = bl1.reshape(1, D)
    bl2_2 = bl2.reshape(1, D)
    bl3_2 = bl3.reshape(1, D)
    Wl1aT, Wl1bT = Wl1[:, :H].T, Wl1[:, H:].T
    Wl2aT, Wl2bT = Wl2[:, :H].T, Wl2[:, H:].T
    Wl3aT, Wl3bT = Wl3[:, :H].T, Wl3[:, H:].T
    Wr2aT, Wr2bT = Wr2[:, :H].T, Wr2[:, H:].T
    Wr3aT, Wr3bT = Wr3[:, :H].T, Wr3[:, H:].T
    a1 = a.reshape(1).astype(f32)

    cnt32 = _sc_counts(dst)
    inv8 = _tc_inv(cnt32)

    sA, sB = _sc_agg(xA, xB, src, dst, zeros)
    z2A, z2B = _tc1(x, sA, sB, inv8, WsT, bs2, Wl1aT, Wl1bT, bl1_2,
                    Wr1T, a1)

    sA, sB = _sc_agg(z2A, z2B, src, dst, zeros)
    z3A, z3B = _tc2(z2A, z2B, sA, sB, inv8, Wl2aT, Wl2bT, bl2_2, Wr2aT,
                    Wr2bT, a1)

    sA, sB = _sc_agg(z3A, z3B, src, dst, zeros)
    h3A, h3B = _tc3(z3A, z3B, sA, sB, inv8, Wl3aT, Wl3bT, bl3_2, Wr3aT,
                    Wr3bT, a1)

    return jnp.concatenate([h3A, h3B], axis=1)


# speedup vs baseline: 8.1529x; 1.0025x over previous
import functools

import jax
import jax.numpy as jnp
from jax import lax
from jax.experimental import pallas as pl
from jax.experimental.pallas import tpu as pltpu
from jax.experimental.pallas import tpu_sc as plsc

N = 10000
E = 160000
D = 256
H = 128
NC = 2
NS = 16
K = 40
EPW = E // NS
NCHUNK = EPW // K
NP = 10240
RPW = NP // NS

_mesh = plsc.VectorSubcoreMesh(core_axis_name="c", subcore_axis_name="s")


G = 5
NOUTER = NCHUNK // G


@functools.partial(
    pl.kernel,
    out_type=(
        jax.ShapeDtypeStruct((NP, H), jnp.float32),
        jax.ShapeDtypeStruct((NP, H), jnp.float32),
    ),
    mesh=_mesh,
    scratch_types=[
        pltpu.VMEM_SHARED((NP, H), jnp.float32),
        pltpu.VMEM((2, G, K), jnp.int32),
        pltpu.VMEM((2, G, K), jnp.int32),
        pltpu.VMEM((G, K, H), jnp.float32),
        [pltpu.SemaphoreType.DMA] * (2 * G),
        [pltpu.SemaphoreType.DMA] * (2 * G),
        [pltpu.SemaphoreType.DMA] * G,
        [pltpu.SemaphoreType.DMA] * G,
    ],
    compiler_params=pltpu.CompilerParams(skip_device_barrier=True),
)
def _sc_agg(zA, zB, src, dst, zeros, outA, outB, acc, idx_s, idx_d, rows,
            isems, dsems, gsems, ssems):
    c = lax.axis_index("c")
    s = lax.axis_index("s")
    r0 = pl.multiple_of(s * RPW, 8)
    pltpu.sync_copy(zeros.at[pl.ds(r0, RPW)], acc.at[pl.ds(r0, RPW)])
    plsc.subcore_barrier()

    def issue_idx(jo, p, b):
        base = pl.multiple_of(s * EPW + (jo * G + b) * K, 8)
        pltpu.async_copy(src.at[pl.ds(base, K)], idx_s.at[p, b],
                         isems[p * G + b])
        pltpu.async_copy(dst.at[pl.ds(base, K)], idx_d.at[p, b],
                         dsems[p * G + b])

    def wait_idx_s(p, b):
        pltpu.make_async_copy(src.at[pl.ds(0, K)], idx_s.at[p, b],
                              isems[p * G + b]).wait()

    def wait_idx_d(p, b):
        pltpu.make_async_copy(dst.at[pl.ds(0, K)], idx_d.at[p, b],
                              dsems[p * G + b]).wait()

    def wait_scatter(p, b):
        pltpu.make_async_copy(rows.at[b], acc.at[idx_d.at[p, b]],
                              ssems[b]).wait()

    def make_body(z_ref):
        def body(jo, p):
            q = 1 - p
            gds = []
            for b in range(G):
                @pl.when(jo > 0)
                def _():
                    wait_scatter(q, b)

                @pl.when(jo + 1 < NOUTER)
                def _():
                    issue_idx(jo + 1, q, b)

                wait_idx_s(p, b)
                gds.append(pltpu.async_copy(z_ref.at[idx_s.at[p, b]],
                                            rows.at[b], gsems[b]))
            for b in range(G):
                gds[b].wait()
                wait_idx_d(p, b)
                pltpu.async_copy(rows.at[b], acc.at[idx_d.at[p, b]],
                                 ssems[b], add=True)
        return body

    def make_outer(z_ref):
        body = make_body(z_ref)

        def outer(jo2, carry):
            body(2 * jo2, 0)
            body(2 * jo2 + 1, 1)
            return carry
        return outer

    def run(z_ref):
        for b in range(G):
            issue_idx(0, 0, b)
        lax.fori_loop(0, NOUTER // 2, make_outer(z_ref), 0)
        for b in range(G):
            wait_scatter(1, b)

    @pl.when(c == 0)
    def _():
        run(zA)

    @pl.when(c == 1)
    def _():
        run(zB)

    plsc.subcore_barrier()

    @pl.when(c == 0)
    def _():
        pltpu.sync_copy(acc.at[pl.ds(r0, RPW)], outA.at[pl.ds(r0, RPW)])

    @pl.when(c == 1)
    def _():
        pltpu.sync_copy(acc.at[pl.ds(r0, RPW)], outB.at[pl.ds(r0, RPW)])


EPW2 = E // (NC * NS)
NVEC2 = EPW2 // 16


@functools.partial(
    pl.kernel,
    out_type=jax.ShapeDtypeStruct((NC * NS, NP), jnp.float32),
    mesh=_mesh,
    scratch_types=[
        pltpu.VMEM((NP,), jnp.float32),
        pltpu.VMEM((EPW2,), jnp.int32),
        pltpu.SemaphoreType.DMA,
    ],
    compiler_params=pltpu.CompilerParams(needs_layout_passes=False,
                                         skip_device_barrier=True),
)
def _sc_counts(dst, out, hist, idxall, sem):
    c = lax.axis_index("c")
    s = lax.axis_index("s")
    w = c * NS + s
    base = pl.multiple_of(w * EPW2, 8)
    pltpu.sync_copy(dst.at[pl.ds(base, EPW2)], idxall)

    zero16 = jnp.zeros((16,), jnp.float32)

    def zero_step(i, carry):
        hist[pl.ds(i * 16, 16)] = zero16
        return carry

    lax.fori_loop(0, NP // 16, zero_step, 0)

    ones = jnp.ones((16,), jnp.float32)

    def add_step(i, carry):
        idx = idxall[pl.ds(i * 16, 16)]
        plsc.addupdate_scatter(hist, [idx], ones)
        return carry

    lax.fori_loop(0, NVEC2, add_step, 0)
    tail = idxall[pl.ds(NVEC2 * 16 - 8, 16)]
    mask = lax.iota(jnp.int32, 16) >= 8
    plsc.addupdate_scatter(hist, [tail], ones, mask=mask)

    pltpu.sync_copy(hist, out.at[w])


def _inv_body(cnt_ref, out_ref):
    t = jnp.transpose(cnt_ref[...])
    cnt = jnp.sum(t, axis=1, keepdims=True)
    inv = 1.0 / jnp.maximum(cnt, 1.0)
    out_ref[...] = jnp.broadcast_to(inv, out_ref.shape)


R = 1024
GRID = NP // R

_row_spec_h = pl.BlockSpec((R, H), lambda i: (i, 0))
_row_spec_d = pl.BlockSpec((R, D), lambda i: (i, 0))
_cnt_spec = pl.BlockSpec((R, 8), lambda i: (i, 0))
_w_hd = pl.BlockSpec((H, D), lambda i: (0, 0))
_w_dd = pl.BlockSpec((D, D), lambda i: (0, 0))
_b_spec = pl.BlockSpec((1, D), lambda i: (0, 0))
_a_spec = pl.BlockSpec(memory_space=pltpu.SMEM)


_tc_inv = pl.pallas_call(
    _inv_body,
    grid=(GRID,),
    in_specs=[pl.BlockSpec((NC * NS, R), lambda i: (0, i))],
    out_specs=pl.BlockSpec((R, 8), lambda i: (i, 0)),
    out_shape=jax.ShapeDtypeStruct((NP, 8), jnp.float32),
)


def _prelu(v, a):
    return jnp.where(v >= 0, v, a * v)


def _mean_term(sA_ref, sB_ref, inv_ref, WlaT_ref, WlbT_ref, bl_ref):
    inv = inv_ref[:, 0:1]
    s = (
        jnp.dot(sA_ref[...], WlaT_ref[...], preferred_element_type=jnp.float32)
        + jnp.dot(sB_ref[...], WlbT_ref[...], preferred_element_type=jnp.float32)
    )
    return s * inv + bl_ref[...]


def _tc1_body(x_ref, sA_ref, sB_ref, inv_ref, WsT_ref, bs_ref,
              WlaT_ref, WlbT_ref, bl_ref, WrT_ref, a_ref, outA_ref, outB_ref):
    a = a_ref[0]
    x = x_ref[...]
    root = jnp.dot(x, WrT_ref[...], preferred_element_type=jnp.float32)
    h1 = _prelu(_mean_term(sA_ref, sB_ref, inv_ref, WlaT_ref,
                           WlbT_ref, bl_ref) + root, a)
    z2 = jnp.dot(x, WsT_ref[...], preferred_element_type=jnp.float32) \
        + bs_ref[...] + h1
    outA_ref[...] = z2[:, :H]
    outB_ref[...] = z2[:, H:]


_tc1 = pl.pallas_call(
    _tc1_body,
    grid=(GRID,),
    in_specs=[_row_spec_d, _row_spec_h, _row_spec_h, _cnt_spec,
              _w_dd, _b_spec, _w_hd, _w_hd, _b_spec, _w_dd, _a_spec],
    out_specs=(_row_spec_h, _row_spec_h),
    out_shape=(
        jax.ShapeDtypeStruct((N, H), jnp.float32),
        jax.ShapeDtypeStruct((N, H), jnp.float32),
    ),
)


def _tc23_body(residual, zA_ref, zB_ref, sA_ref, sB_ref, inv_ref,
               WlaT_ref, WlbT_ref, bl_ref, WraT_ref, WrbT_ref, a_ref,
               outA_ref, outB_ref):
    a = a_ref[0]
    root = (
        jnp.dot(zA_ref[...], WraT_ref[...], preferred_element_type=jnp.float32)
        + jnp.dot(zB_ref[...], WrbT_ref[...], preferred_element_type=jnp.float32)
    )
    h = _prelu(_mean_term(sA_ref, sB_ref, inv_ref, WlaT_ref,
                          WlbT_ref, bl_ref) + root, a)
    if residual:
        outA_ref[...] = zA_ref[...] + h[:, :H]
        outB_ref[...] = zB_ref[...] + h[:, H:]
    else:
        outA_ref[...] = h[:, :H]
        outB_ref[...] = h[:, H:]


def _make_tc23(residual):
    return pl.pallas_call(
        functools.partial(_tc23_body, residual),
        grid=(GRID,),
        in_specs=[_row_spec_h, _row_spec_h, _row_spec_h, _row_spec_h,
                  _cnt_spec, _w_hd, _w_hd, _b_spec, _w_hd,
                  _w_hd, _a_spec],
        out_specs=(_row_spec_h, _row_spec_h),
        out_shape=(
            jax.ShapeDtypeStruct((N, H), jnp.float32),
            jax.ShapeDtypeStruct((N, H), jnp.float32),
        ),
    )


_tc2 = _make_tc23(True)
_tc3 = _make_tc23(False)


def kernel(x, W_skip, b_skip, Wl1, bl1, Wr1, Wl2, bl2, Wr2, Wl3, bl3, Wr3, a,
           edge_index):
    f32 = jnp.float32
    src = edge_index[0].astype(jnp.int32)
    dst = edge_index[1].astype(jnp.int32)

    xA = x[:, :H]
    xB = x[:, H:]
    zeros = jnp.zeros((NP, H), f32)

    WsT = W_skip.T
    Wr1T = Wr1.T
    bs2 = b_skip.reshape(1, D)
    bl1_2 = bl1.reshape(1, D)
    bl2_2 = bl2.reshape(1, D)
    bl3_2 = bl3.reshape(1, D)
    Wl1aT, Wl1bT = Wl1[:, :H].T, Wl1[:, H:].T
    Wl2aT, Wl2bT = Wl2[:, :H].T, Wl2[:, H:].T
    Wl3aT, Wl3bT = Wl3[:, :H].T, Wl3[:, H:].T
    Wr2aT, Wr2bT = Wr2[:, :H].T, Wr2[:, H:].T
    Wr3aT, Wr3bT = Wr3[:, :H].T, Wr3[:, H:].T
    a1 = a.reshape(1).astype(f32)

    cnt32 = _sc_counts(dst)
    inv8 = _tc_inv(cnt32)

    sA, sB = _sc_agg(xA, xB, src, dst, zeros)
    z2A, z2B = _tc1(x, sA, sB, inv8, WsT, bs2, Wl1aT, Wl1bT, bl1_2,
                    Wr1T, a1)

    sA, sB = _sc_agg(z2A, z2B, src, dst, zeros)
    z3A, z3B = _tc2(z2A, z2B, sA, sB, inv8, Wl2aT, Wl2bT, bl2_2, Wr2aT,
                    Wr2bT, a1)

    sA, sB = _sc_agg(z3A, z3B, src, dst, zeros)
    h3A, h3B = _tc3(z3A, z3B, sA, sB, inv8, Wl3aT, Wl3bT, bl3_2, Wr3aT,
                    Wr3bT, a1)

    return jnp.concatenate([h3A, h3B], axis=1)
